# Initial kernel scaffold; baseline (speedup 1.0000x reference)
import jax
import jax.numpy as jnp
from jax.experimental import pallas as pl


def kernel(input, weights):
    raise NotImplementedError("write your pallas kernel here")


# speedup vs baseline: 3.1388x; 3.1388x over previous
import jax
import jax.numpy as jnp
from jax.experimental import pallas as pl

_PAD = 1
_T = 256


def _body(tok_ref, w_ref, pad_ref, out_ref):
    w = w_ref[...]
    pad = jnp.broadcast_to(pad_ref[...], w.shape)
    bsz = out_ref.shape[0]
    for b in range(bsz):
        mask = tok_ref[:, b : b + 1] != _PAD
        out_ref[b] = jnp.where(mask, w, pad)


def kernel(input, weights):
    bsz, seq_len = input.shape
    dim = weights.shape[1]
    tok_t = input.T
    w_seq = jax.lax.slice(weights, (2, 0), (2 + seq_len, dim))
    pad_row = jax.lax.slice(weights, (1, 0), (2, dim))
    grid = (seq_len // _T,)
    return pl.pallas_call(
        _body,
        grid=grid,
        in_specs=[
            pl.BlockSpec((_T, bsz), lambda j: (j, 0)),
            pl.BlockSpec((_T, dim), lambda j: (j, 0)),
            pl.BlockSpec((1, dim), lambda j: (0, 0)),
        ],
        out_specs=pl.BlockSpec((bsz, _T, dim), lambda j: (0, j, 0)),
        out_shape=jax.ShapeDtypeStruct((bsz, seq_len, dim), weights.dtype),
    )(tok_t, w_seq, pad_row)


# speedup vs baseline: 4.5007x; 1.4339x over previous
import jax
import jax.numpy as jnp
from jax.experimental import pallas as pl
from jax.experimental.pallas import tpu as pltpu

_PAD = 1
_T = 256


def _body(tok_ref, w_ref, out_ref):
    j = pl.program_id(0)
    w_ext = w_ref[pl.ds(j * _T, _T + 8), :]
    w = pltpu.roll(w_ext, _T + 6, 0)[:_T, :]
    pad = jnp.broadcast_to(w_ref[0:8, :][_PAD : _PAD + 1, :], w.shape)
    bsz = out_ref.shape[0]
    for b in range(bsz):
        mask = tok_ref[:, b : b + 1] != _PAD
        out_ref[b] = jnp.where(mask, w, pad)


def kernel(input, weights):
    bsz, seq_len = input.shape
    num_rows, dim = weights.shape
    tok_t = input.T
    grid = (seq_len // _T,)
    return pl.pallas_call(
        _body,
        grid=grid,
        in_specs=[
            pl.BlockSpec((_T, bsz), lambda j: (j, 0)),
            pl.BlockSpec((num_rows, dim), lambda j: (0, 0)),
        ],
        out_specs=pl.BlockSpec((bsz, _T, dim), lambda j: (0, j, 0)),
        out_shape=jax.ShapeDtypeStruct((bsz, seq_len, dim), weights.dtype),
    )(tok_t, weights)
